# baseline (speedup 1.0000x reference)
import functools

import jax
import jax.numpy as jnp
import numpy as np
from jax import lax
from jax.experimental import pallas as pl
from jax.experimental.pallas import tpu as pltpu
from jax.experimental.pallas import tpu_sc as plsc

N_ATOMS = 100000
N_DIH = 500000
NC = 2
NS = 16
L = 16
NW = NC * NS
PER_W = 16384
NPAD = NW * PER_W
BLK = 128
NBLK_CHUNK = 8
CHUNK = BLK * NBLK_CHUNK
NCHUNK = PER_W // CHUNK
GROUPS = CHUNK // L
NSTR = 12

_SIN = (1.0, -1.6666651e-01, 8.3329640e-03, -1.9804748e-04, 2.5980951e-06)
_COS = (1.0, -0.5, 4.1666642e-02, -1.3888433e-03, 2.4763767e-05, -2.6114949e-07)
_HALF_PI = np.float32(1.5707964)


def _rsqrt(q):
    xi = lax.bitcast_convert_type(q, jnp.int32)
    yi = jnp.int32(0x5F3759DF) - lax.shift_right_logical(xi, 1)
    y = lax.bitcast_convert_type(yi, jnp.float32)
    h = q * np.float32(0.5)
    y = y * (np.float32(1.5) - h * y * y)
    y = y * (np.float32(1.5) - h * y * y)
    return y


def _sc_body(cflat_ref, idx_ref, f_ref, per_ref, ph_ref,
             out_ref, idx_v, f_v, per_v, ph_v, gbuf, acc_v, sem):
    cidx = lax.axis_index("c")
    sidx = lax.axis_index("s")
    wid = sidx * NC + cidx
    blk0 = wid * (PER_W // BLK)
    el0 = wid * PER_W
    f1 = np.float32(1.0)

    def load_and_fire(cc, buf):
        row0 = blk0 + cc * NBLK_CHUNK
        e0 = el0 + cc * CHUNK
        pltpu.sync_copy(
            idx_ref.at[pl.ds(row0 * NSTR, NBLK_CHUNK * NSTR)],
            idx_v.at[pl.ds(buf * NBLK_CHUNK * NSTR, NBLK_CHUNK * NSTR)])
        pltpu.sync_copy(f_ref.at[pl.ds(e0, CHUNK)], f_v.at[buf])
        pltpu.sync_copy(per_ref.at[pl.ds(e0, CHUNK)], per_v.at[buf])
        pltpu.sync_copy(ph_ref.at[pl.ds(e0, CHUNK)], ph_v.at[buf])

        @pl.loop(0, NBLK_CHUNK)
        def _fire(b):
            dsts = pl.ds(b * BLK, BLK)
            for c in range(NSTR):
                pltpu.async_copy(
                    cflat_ref.at[idx_v.at[buf * NBLK_CHUNK * NSTR
                                          + b * NSTR + c]],
                    gbuf.at[buf * NSTR + c, dsts], sem)

    def process(cc, buf, acc):
        for c in range(NSTR):
            pltpu.make_async_copy(
                cflat_ref.at[pl.ds(0, CHUNK)], gbuf.at[buf * NSTR + c],
                sem).wait()

        @pl.when(cc + 1 < NCHUNK)
        def _prefetch():
            load_and_fire(cc + 1, 1 - buf)

        def group_body(g, acc):
            base = g * L
            sl = pl.ds(base, L)
            p0x = gbuf[buf * NSTR + 0, sl]
            p0y = gbuf[buf * NSTR + 1, sl]
            p0z = gbuf[buf * NSTR + 2, sl]
            p1x = gbuf[buf * NSTR + 3, sl]
            p1y = gbuf[buf * NSTR + 4, sl]
            p1z = gbuf[buf * NSTR + 5, sl]
            p2x = gbuf[buf * NSTR + 6, sl]
            p2y = gbuf[buf * NSTR + 7, sl]
            p2z = gbuf[buf * NSTR + 8, sl]
            p3x = gbuf[buf * NSTR + 9, sl]
            p3y = gbuf[buf * NSTR + 10, sl]
            p3z = gbuf[buf * NSTR + 11, sl]

            v1x = p0x - p1x
            v1y = p0y - p1y
            v1z = p0z - p1z
            v2x = p2x - p1x
            v2y = p2y - p1y
            v2z = p2z - p1z
            v3x = p2x - p3x
            v3y = p2y - p3y
            v3z = p2z - p3z

            c12x = v1y * v2z - v1z * v2y
            c12y = v1z * v2x - v1x * v2z
            c12z = v1x * v2y - v1y * v2x
            c23x = v2y * v3z - v2z * v3y
            c23y = v2z * v3x - v2x * v3z
            c23z = v2x * v3y - v2y * v3x

            a2 = c12x * c12x + c12y * c12y + c12z * c12z
            b2 = c23x * c23x + c23y * c23y + c23z * c23z
            dd = c12x * c23x + c12y * c23y + c12z * c23z
            tt = v1x * c23x + v1y * c23y + v1z * c23z

            q = jnp.maximum(a2 * b2, np.float32(1e-24))
            r = _rsqrt(q)
            c = jnp.clip(dd * r, np.float32(-1.0), np.float32(1.0))
            om = f1 - c * c
            sm = om * _rsqrt(jnp.maximum(om, np.float32(1e-30)))
            s = jnp.where(tt < np.float32(0.0), -sm, sm)

            fv = f_v[buf, sl]
            pv = per_v[buf, sl]
            phv = ph_v[buf, sl]

            t = phv - _HALF_PI
            t2 = t * t
            sp = np.float32(_SIN[4])
            for cf in (_SIN[3], _SIN[2], _SIN[1], _SIN[0]):
                sp = sp * t2 + np.float32(cf)
            sp = sp * t
            cp = np.float32(_COS[5])
            for cf in (_COS[4], _COS[3], _COS[2], _COS[1], _COS[0]):
                cp = cp * t2 + np.float32(cf)
            cpsi = -sp
            spsi = cp

            cn, sn = c, s
            ck, sk = c, s
            for kk2 in range(2, 7):
                ck, sk = ck * c - sk * s, sk * c + ck * s
                sel = pv == np.float32(kk2)
                cn = jnp.where(sel, ck, cn)
                sn = jnp.where(sel, sk, sn)

            val = fv * (f1 + cn * cpsi + sn * spsi)
            return acc + val

        return lax.fori_loop(0, GROUPS, group_body, acc, unroll=False)

    load_and_fire(0, 0)

    def pair_body(p, acc):
        acc = process(2 * p, 0, acc)
        acc = process(2 * p + 1, 1, acc)
        return acc

    acc = lax.fori_loop(0, NCHUNK // 2, pair_body,
                        jnp.zeros((L,), jnp.float32), unroll=False)
    acc_v[...] = acc
    pltpu.sync_copy(acc_v, out_ref.at[wid])


_sc_call = functools.partial(
    pl.kernel,
    out_type=jax.ShapeDtypeStruct((NW, L), jnp.float32),
    mesh=plsc.VectorSubcoreMesh(core_axis_name="c", subcore_axis_name="s",
                                num_cores=NC, num_subcores=NS),
    scratch_types=[
        pltpu.VMEM((2 * NBLK_CHUNK * NSTR, BLK), jnp.int32),
        pltpu.VMEM((2, CHUNK), jnp.float32),
        pltpu.VMEM((2, CHUNK), jnp.float32),
        pltpu.VMEM((2, CHUNK), jnp.float32),
        pltpu.VMEM((2 * NSTR, CHUNK), jnp.float32),
        pltpu.VMEM((L,), jnp.float32),
        pltpu.SemaphoreType.DMA,
    ],
)(_sc_body)


@jax.jit
def kernel(coords, i, j, k, l, force, period, phase):
    cflat = coords.reshape(N_ATOMS * 3)
    pad = NPAD - N_DIH

    def pad_i32(x, val):
        return jnp.concatenate(
            [x.astype(jnp.int32), jnp.full((pad,), val, jnp.int32)])

    streams = []
    for idx in (i, j, k, l):
        base3 = pad_i32(idx, 0) * 3
        streams.extend([base3, base3 + 1, base3 + 2])
    idx_all = jnp.stack(streams)
    idx_all = (idx_all.reshape(NSTR, NPAD // BLK, BLK).transpose(1, 0, 2)
               .reshape(NPAD // BLK * NSTR, BLK))

    f2 = jnp.concatenate([force, jnp.zeros((pad,), jnp.float32)])
    p2 = jnp.concatenate([period, jnp.ones((pad,), jnp.float32)])
    ph2 = jnp.concatenate([phase, jnp.zeros((pad,), jnp.float32)])
    partials = _sc_call(cflat, idx_all, f2, p2, ph2)
    return jnp.sum(partials)


# speedup vs baseline: 4.4933x; 4.4933x over previous
import functools

import jax
import jax.numpy as jnp
import numpy as np
from jax import lax
from jax.experimental import pallas as pl
from jax.experimental.pallas import tpu as pltpu
from jax.experimental.pallas import tpu_sc as plsc

N_ATOMS = 100000
N_DIH = 500000
NC = 2
NS = 16
L = 16
NW = NC * NS
PER_W = 16384
NPAD = NW * PER_W
BLK = 128
NBLK_CHUNK = 8
CHUNK = BLK * NBLK_CHUNK
NCHUNK = PER_W // CHUNK
GROUPS = CHUNK // L
NSTR = 12
CSTAGE = 300032
CSLICE = CSTAGE // NS

_SIN = (1.0, -1.6666651e-01, 8.3329640e-03, -1.9804748e-04, 2.5980951e-06)
_COS = (1.0, -0.5, 4.1666642e-02, -1.3888433e-03, 2.4763767e-05, -2.6114949e-07)
_HALF_PI = np.float32(1.5707964)


def _rsqrt(q):
    xi = lax.bitcast_convert_type(q, jnp.int32)
    yi = jnp.int32(0x5F3759DF) - lax.shift_right_logical(xi, 1)
    y = lax.bitcast_convert_type(yi, jnp.float32)
    h = q * np.float32(0.5)
    y = y * (np.float32(1.5) - h * y * y)
    y = y * (np.float32(1.5) - h * y * y)
    return y


def _sc_body(cflat_ref, idx_ref, f_ref, per_ref, ph_ref,
             out_ref, idx_v, f_v, per_v, ph_v, gbuf, acc_v, stage_v,
             cshared, sem):
    cidx = lax.axis_index("c")
    sidx = lax.axis_index("s")
    wid = sidx * NC + cidx
    blk0 = wid * (PER_W // BLK)
    el0 = wid * PER_W
    f1 = np.float32(1.0)

    s0 = sidx * CSLICE
    pltpu.sync_copy(cflat_ref.at[pl.ds(s0, CSLICE)], stage_v)
    pltpu.sync_copy(stage_v, cshared.at[pl.ds(s0, CSLICE)])
    plsc.subcore_barrier()

    def load_and_fire(cc, buf):
        row0 = blk0 + cc * NBLK_CHUNK
        e0 = el0 + cc * CHUNK
        pltpu.sync_copy(
            idx_ref.at[pl.ds(row0 * NSTR, NBLK_CHUNK * NSTR)],
            idx_v.at[pl.ds(buf * NBLK_CHUNK * NSTR, NBLK_CHUNK * NSTR)])
        pltpu.sync_copy(f_ref.at[pl.ds(e0, CHUNK)], f_v.at[buf])
        pltpu.sync_copy(per_ref.at[pl.ds(e0, CHUNK)], per_v.at[buf])
        pltpu.sync_copy(ph_ref.at[pl.ds(e0, CHUNK)], ph_v.at[buf])

        @pl.loop(0, NBLK_CHUNK)
        def _fire(b):
            dsts = pl.ds(b * BLK, BLK)
            for c in range(NSTR):
                pltpu.async_copy(
                    cshared.at[idx_v.at[buf * NBLK_CHUNK * NSTR
                                        + b * NSTR + c]],
                    gbuf.at[buf * NSTR + c, dsts], sem)

    def process(cc, buf, acc):
        for c in range(NSTR):
            pltpu.make_async_copy(
                cflat_ref.at[pl.ds(0, CHUNK)], gbuf.at[buf * NSTR + c],
                sem).wait()

        @pl.when(cc + 1 < NCHUNK)
        def _prefetch():
            load_and_fire(cc + 1, 1 - buf)

        def group_body(g, acc):
            base = g * L
            sl = pl.ds(base, L)
            p0x = gbuf[buf * NSTR + 0, sl]
            p0y = gbuf[buf * NSTR + 1, sl]
            p0z = gbuf[buf * NSTR + 2, sl]
            p1x = gbuf[buf * NSTR + 3, sl]
            p1y = gbuf[buf * NSTR + 4, sl]
            p1z = gbuf[buf * NSTR + 5, sl]
            p2x = gbuf[buf * NSTR + 6, sl]
            p2y = gbuf[buf * NSTR + 7, sl]
            p2z = gbuf[buf * NSTR + 8, sl]
            p3x = gbuf[buf * NSTR + 9, sl]
            p3y = gbuf[buf * NSTR + 10, sl]
            p3z = gbuf[buf * NSTR + 11, sl]

            v1x = p0x - p1x
            v1y = p0y - p1y
            v1z = p0z - p1z
            v2x = p2x - p1x
            v2y = p2y - p1y
            v2z = p2z - p1z
            v3x = p2x - p3x
            v3y = p2y - p3y
            v3z = p2z - p3z

            c12x = v1y * v2z - v1z * v2y
            c12y = v1z * v2x - v1x * v2z
            c12z = v1x * v2y - v1y * v2x
            c23x = v2y * v3z - v2z * v3y
            c23y = v2z * v3x - v2x * v3z
            c23z = v2x * v3y - v2y * v3x

            a2 = c12x * c12x + c12y * c12y + c12z * c12z
            b2 = c23x * c23x + c23y * c23y + c23z * c23z
            dd = c12x * c23x + c12y * c23y + c12z * c23z
            tt = v1x * c23x + v1y * c23y + v1z * c23z

            q = jnp.maximum(a2 * b2, np.float32(1e-24))
            r = _rsqrt(q)
            c = jnp.clip(dd * r, np.float32(-1.0), np.float32(1.0))
            om = f1 - c * c
            sm = om * _rsqrt(jnp.maximum(om, np.float32(1e-30)))
            s = jnp.where(tt < np.float32(0.0), -sm, sm)

            fv = f_v[buf, sl]
            pv = per_v[buf, sl]
            phv = ph_v[buf, sl]

            t = phv - _HALF_PI
            t2 = t * t
            sp = np.float32(_SIN[4])
            for cf in (_SIN[3], _SIN[2], _SIN[1], _SIN[0]):
                sp = sp * t2 + np.float32(cf)
            sp = sp * t
            cp = np.float32(_COS[5])
            for cf in (_COS[4], _COS[3], _COS[2], _COS[1], _COS[0]):
                cp = cp * t2 + np.float32(cf)
            cpsi = -sp
            spsi = cp

            cn, sn = c, s
            ck, sk = c, s
            for kk2 in range(2, 7):
                ck, sk = ck * c - sk * s, sk * c + ck * s
                sel = pv == np.float32(kk2)
                cn = jnp.where(sel, ck, cn)
                sn = jnp.where(sel, sk, sn)

            val = fv * (f1 + cn * cpsi + sn * spsi)
            return acc + val

        return lax.fori_loop(0, GROUPS, group_body, acc, unroll=False)

    load_and_fire(0, 0)

    def pair_body(p, acc):
        acc = process(2 * p, 0, acc)
        acc = process(2 * p + 1, 1, acc)
        return acc

    acc = lax.fori_loop(0, NCHUNK // 2, pair_body,
                        jnp.zeros((L,), jnp.float32), unroll=False)
    acc_v[...] = acc
    pltpu.sync_copy(acc_v, out_ref.at[wid])


_sc_call = functools.partial(
    pl.kernel,
    out_type=jax.ShapeDtypeStruct((NW, L), jnp.float32),
    mesh=plsc.VectorSubcoreMesh(core_axis_name="c", subcore_axis_name="s",
                                num_cores=NC, num_subcores=NS),
    scratch_types=[
        pltpu.VMEM((2 * NBLK_CHUNK * NSTR, BLK), jnp.int32),
        pltpu.VMEM((2, CHUNK), jnp.float32),
        pltpu.VMEM((2, CHUNK), jnp.float32),
        pltpu.VMEM((2, CHUNK), jnp.float32),
        pltpu.VMEM((2 * NSTR, CHUNK), jnp.float32),
        pltpu.VMEM((L,), jnp.float32),
        pltpu.VMEM((CSLICE,), jnp.float32),
        pltpu.VMEM_SHARED((CSTAGE,), jnp.float32),
        pltpu.SemaphoreType.DMA,
    ],
)(_sc_body)


@jax.jit
def kernel(coords, i, j, k, l, force, period, phase):
    cflat = jnp.concatenate(
        [coords.reshape(N_ATOMS * 3),
         jnp.zeros((CSTAGE - N_ATOMS * 3,), jnp.float32)])
    pad = NPAD - N_DIH

    def pad_i32(x, val):
        return jnp.concatenate(
            [x.astype(jnp.int32), jnp.full((pad,), val, jnp.int32)])

    streams = []
    for idx in (i, j, k, l):
        base3 = pad_i32(idx, 0) * 3
        streams.extend([base3, base3 + 1, base3 + 2])
    idx_all = jnp.stack(streams)
    idx_all = (idx_all.reshape(NSTR, NPAD // BLK, BLK).transpose(1, 0, 2)
               .reshape(NPAD // BLK * NSTR, BLK))

    f2 = jnp.concatenate([force, jnp.zeros((pad,), jnp.float32)])
    p2 = jnp.concatenate([period, jnp.ones((pad,), jnp.float32)])
    ph2 = jnp.concatenate([phase, jnp.zeros((pad,), jnp.float32)])
    partials = _sc_call(cflat, idx_all, f2, p2, ph2)
    return jnp.sum(partials)


# speedup vs baseline: 5.9948x; 1.3342x over previous
import functools

import jax
import jax.numpy as jnp
import numpy as np
from jax import lax
from jax.experimental import pallas as pl
from jax.experimental.pallas import tpu as pltpu
from jax.experimental.pallas import tpu_sc as plsc

N_ATOMS = 100000
N_DIH = 500000
NC = 2
NS = 16
L = 16
NW = NC * NS
PER_W = 16384
NPAD = NW * PER_W
BLK = 128
NBLK_CHUNK = 8
CHUNK = BLK * NBLK_CHUNK
NCHUNK = PER_W // CHUNK
GROUPS = CHUNK // L
NSTR = 12
CSTAGE = 300032
CSLICE = CSTAGE // NS

_SIN = (1.0, -1.6666651e-01, 8.3329640e-03, -1.9804748e-04, 2.5980951e-06)
_COS = (1.0, -0.5, 4.1666642e-02, -1.3888433e-03, 2.4763767e-05, -2.6114949e-07)
_HALF_PI = np.float32(1.5707964)


def _rsqrt(q):
    xi = lax.bitcast_convert_type(q, jnp.int32)
    yi = jnp.int32(0x5F3759DF) - lax.shift_right_logical(xi, 1)
    y = lax.bitcast_convert_type(yi, jnp.float32)
    h = q * np.float32(0.5)
    y = y * (np.float32(1.5) - h * y * y)
    y = y * (np.float32(1.5) - h * y * y)
    return y


def _sc_body(cflat_ref, i_ref, j_ref, k_ref, l_ref, f_ref, per_ref, ph_ref,
             out_ref, raw_v, idx_v, f_v, per_v, ph_v, gbuf, acc_v, stage_v,
             cshared, sem):
    cidx = lax.axis_index("c")
    sidx = lax.axis_index("s")
    wid = sidx * NC + cidx
    blk0 = wid * (PER_W // BLK)
    el0 = wid * PER_W
    f1 = np.float32(1.0)

    s0 = sidx * CSLICE
    pltpu.sync_copy(cflat_ref.at[pl.ds(s0, CSLICE)], stage_v)
    pltpu.sync_copy(stage_v, cshared.at[pl.ds(s0, CSLICE)])
    plsc.subcore_barrier()

    def load_and_fire(cc, buf):
        row0 = blk0 + cc * NBLK_CHUNK
        e0 = el0 + cc * CHUNK
        for r, ref in enumerate((i_ref, j_ref, k_ref, l_ref)):
            pltpu.sync_copy(
                ref.at[pl.ds(row0, NBLK_CHUNK)],
                raw_v.at[pl.ds((buf * 4 + r) * NBLK_CHUNK, NBLK_CHUNK)])
        pltpu.sync_copy(f_ref.at[pl.ds(e0, CHUNK)], f_v.at[buf])
        pltpu.sync_copy(per_ref.at[pl.ds(e0, CHUNK)], per_v.at[buf])
        pltpu.sync_copy(ph_ref.at[pl.ds(e0, CHUNK)], ph_v.at[buf])

        @pl.loop(0, NBLK_CHUNK)
        def _expand(b):
            for r in range(4):
                rrow = (buf * 4 + r) * NBLK_CHUNK + b
                for g in range(BLK // L):
                    sl16 = pl.ds(g * L, L)
                    v3 = raw_v[rrow, sl16] * 3
                    srow = buf * NBLK_CHUNK * NSTR + b * NSTR + r * 3
                    idx_v[srow, sl16] = v3
                    idx_v[srow + 1, sl16] = v3 + 1
                    idx_v[srow + 2, sl16] = v3 + 2

        @pl.loop(0, NBLK_CHUNK)
        def _fire(b):
            dsts = pl.ds(b * BLK, BLK)
            for c in range(NSTR):
                pltpu.async_copy(
                    cshared.at[idx_v.at[buf * NBLK_CHUNK * NSTR
                                        + b * NSTR + c]],
                    gbuf.at[buf * NSTR + c, dsts], sem)

    def process(cc, buf, acc):
        for c in range(NSTR):
            pltpu.make_async_copy(
                cflat_ref.at[pl.ds(0, CHUNK)], gbuf.at[buf * NSTR + c],
                sem).wait()

        @pl.when(cc + 1 < NCHUNK)
        def _prefetch():
            load_and_fire(cc + 1, 1 - buf)

        def group_body(g, acc):
            base = g * L
            sl = pl.ds(base, L)
            p0x = gbuf[buf * NSTR + 0, sl]
            p0y = gbuf[buf * NSTR + 1, sl]
            p0z = gbuf[buf * NSTR + 2, sl]
            p1x = gbuf[buf * NSTR + 3, sl]
            p1y = gbuf[buf * NSTR + 4, sl]
            p1z = gbuf[buf * NSTR + 5, sl]
            p2x = gbuf[buf * NSTR + 6, sl]
            p2y = gbuf[buf * NSTR + 7, sl]
            p2z = gbuf[buf * NSTR + 8, sl]
            p3x = gbuf[buf * NSTR + 9, sl]
            p3y = gbuf[buf * NSTR + 10, sl]
            p3z = gbuf[buf * NSTR + 11, sl]

            v1x = p0x - p1x
            v1y = p0y - p1y
            v1z = p0z - p1z
            v2x = p2x - p1x
            v2y = p2y - p1y
            v2z = p2z - p1z
            v3x = p2x - p3x
            v3y = p2y - p3y
            v3z = p2z - p3z

            c12x = v1y * v2z - v1z * v2y
            c12y = v1z * v2x - v1x * v2z
            c12z = v1x * v2y - v1y * v2x
            c23x = v2y * v3z - v2z * v3y
            c23y = v2z * v3x - v2x * v3z
            c23z = v2x * v3y - v2y * v3x

            a2 = c12x * c12x + c12y * c12y + c12z * c12z
            b2 = c23x * c23x + c23y * c23y + c23z * c23z
            dd = c12x * c23x + c12y * c23y + c12z * c23z
            tt = v1x * c23x + v1y * c23y + v1z * c23z

            q = jnp.maximum(a2 * b2, np.float32(1e-24))
            r = _rsqrt(q)
            c = jnp.clip(dd * r, np.float32(-1.0), np.float32(1.0))
            om = f1 - c * c
            sm = om * _rsqrt(jnp.maximum(om, np.float32(1e-30)))
            s = jnp.where(tt < np.float32(0.0), -sm, sm)

            fv = f_v[buf, sl]
            pv = per_v[buf, sl]
            phv = ph_v[buf, sl]

            t = phv - _HALF_PI
            t2 = t * t
            sp = np.float32(_SIN[4])
            for cf in (_SIN[3], _SIN[2], _SIN[1], _SIN[0]):
                sp = sp * t2 + np.float32(cf)
            sp = sp * t
            cp = np.float32(_COS[5])
            for cf in (_COS[4], _COS[3], _COS[2], _COS[1], _COS[0]):
                cp = cp * t2 + np.float32(cf)
            cpsi = -sp
            spsi = cp

            cn, sn = c, s
            ck, sk = c, s
            for kk2 in range(2, 7):
                ck, sk = ck * c - sk * s, sk * c + ck * s
                sel = pv == np.float32(kk2)
                cn = jnp.where(sel, ck, cn)
                sn = jnp.where(sel, sk, sn)

            val = fv * (f1 + cn * cpsi + sn * spsi)
            return acc + val

        return lax.fori_loop(0, GROUPS, group_body, acc, unroll=False)

    load_and_fire(0, 0)

    def pair_body(p, acc):
        acc = process(2 * p, 0, acc)
        acc = process(2 * p + 1, 1, acc)
        return acc

    acc = lax.fori_loop(0, NCHUNK // 2, pair_body,
                        jnp.zeros((L,), jnp.float32), unroll=False)
    acc_v[...] = acc
    pltpu.sync_copy(acc_v, out_ref.at[wid])


_sc_call = functools.partial(
    pl.kernel,
    out_type=jax.ShapeDtypeStruct((NW, L), jnp.float32),
    mesh=plsc.VectorSubcoreMesh(core_axis_name="c", subcore_axis_name="s",
                                num_cores=NC, num_subcores=NS),
    scratch_types=[
        pltpu.VMEM((2 * 4 * NBLK_CHUNK, BLK), jnp.int32),
        pltpu.VMEM((2 * NBLK_CHUNK * NSTR, BLK), jnp.int32),
        pltpu.VMEM((2, CHUNK), jnp.float32),
        pltpu.VMEM((2, CHUNK), jnp.float32),
        pltpu.VMEM((2, CHUNK), jnp.float32),
        pltpu.VMEM((2 * NSTR, CHUNK), jnp.float32),
        pltpu.VMEM((L,), jnp.float32),
        pltpu.VMEM((CSLICE,), jnp.float32),
        pltpu.VMEM_SHARED((CSTAGE,), jnp.float32),
        pltpu.SemaphoreType.DMA,
    ],
)(_sc_body)


@jax.jit
def kernel(coords, i, j, k, l, force, period, phase):
    cflat = jnp.concatenate(
        [coords.reshape(N_ATOMS * 3),
         jnp.zeros((CSTAGE - N_ATOMS * 3,), jnp.float32)])
    pad = NPAD - N_DIH

    def pad_i32(x, val):
        return jnp.concatenate(
            [x.astype(jnp.int32), jnp.full((pad,), val, jnp.int32)])

    i2 = pad_i32(i, 0).reshape(NPAD // BLK, BLK)
    j2 = pad_i32(j, 0).reshape(NPAD // BLK, BLK)
    k2 = pad_i32(k, 0).reshape(NPAD // BLK, BLK)
    l2 = pad_i32(l, 0).reshape(NPAD // BLK, BLK)

    f2 = jnp.concatenate([force, jnp.zeros((pad,), jnp.float32)])
    p2 = jnp.concatenate([period, jnp.ones((pad,), jnp.float32)])
    ph2 = jnp.concatenate([phase, jnp.zeros((pad,), jnp.float32)])
    partials = _sc_call(cflat, i2, j2, k2, l2, f2, p2, ph2)
    return jnp.sum(partials)


# speedup vs baseline: 9.6828x; 1.6152x over previous
import functools

import jax
import jax.numpy as jnp
import numpy as np
from jax import lax
from jax.experimental import pallas as pl
from jax.experimental.pallas import tpu as pltpu
from jax.experimental.pallas import tpu_sc as plsc

N_ATOMS = 100000
N_DIH = 500000
NC = 2
NS = 16
L = 16
NW = NC * NS
PER_W = 16384
NPAD = NW * PER_W
BLK = 128
NBLK_CHUNK = 8
CHUNK = BLK * NBLK_CHUNK
NCHUNK = PER_W // CHUNK
GROUPS = CHUNK // L
ATAB = 100096
ASLICE = ATAB // NS

_SIN = (1.0, -1.6666651e-01, 8.3329640e-03, -1.9804748e-04, 2.5980951e-06)
_COS = (1.0, -0.5, 4.1666642e-02, -1.3888433e-03, 2.4763767e-05, -2.6114949e-07)
_HALF_PI = np.float32(1.5707964)
_HI_MASK = jnp.int32(-65536)


def _rsqrt(q):
    xi = lax.bitcast_convert_type(q, jnp.int32)
    yi = jnp.int32(0x5F3759DF) - lax.shift_right_logical(xi, 1)
    y = lax.bitcast_convert_type(yi, jnp.float32)
    h = q * np.float32(0.5)
    y = y * (np.float32(1.5) - h * y * y)
    y = y * (np.float32(1.5) - h * y * y)
    return y


def _sc_body(xy_ref, z_ref, i_ref, j_ref, k_ref, l_ref, f_ref, per_ref,
             ph_ref, out_ref, raw_v, f_v, per_v, ph_v, gbuf, acc_v, stage_v,
             xy_sh, z_sh, sem):
    cidx = lax.axis_index("c")
    sidx = lax.axis_index("s")
    wid = sidx * NC + cidx
    blk0 = wid * (PER_W // BLK)
    el0 = wid * PER_W
    f1 = np.float32(1.0)

    s0 = sidx * ASLICE
    pltpu.sync_copy(xy_ref.at[pl.ds(s0, ASLICE)], stage_v)
    pltpu.sync_copy(stage_v, xy_sh.at[pl.ds(s0, ASLICE)])
    pltpu.sync_copy(z_ref.at[pl.ds(s0, ASLICE)], stage_v)
    pltpu.sync_copy(stage_v, z_sh.at[pl.ds(s0, ASLICE)])
    plsc.subcore_barrier()

    def load_and_fire(cc, buf):
        row0 = blk0 + cc * NBLK_CHUNK
        e0 = el0 + cc * CHUNK
        for r, ref in enumerate((i_ref, j_ref, k_ref, l_ref)):
            pltpu.sync_copy(
                ref.at[pl.ds(row0, NBLK_CHUNK)],
                raw_v.at[pl.ds((buf * 4 + r) * NBLK_CHUNK, NBLK_CHUNK)])
        pltpu.sync_copy(f_ref.at[pl.ds(e0, CHUNK)], f_v.at[buf])
        pltpu.sync_copy(per_ref.at[pl.ds(e0, CHUNK)], per_v.at[buf])
        pltpu.sync_copy(ph_ref.at[pl.ds(e0, CHUNK)], ph_v.at[buf])

        @pl.loop(0, NBLK_CHUNK)
        def _fire(b):
            dsts = pl.ds(b * BLK, BLK)
            for r in range(4):
                idxrow = raw_v.at[(buf * 4 + r) * NBLK_CHUNK + b]
                pltpu.async_copy(
                    xy_sh.at[idxrow], gbuf.at[buf * 8 + 2 * r, dsts], sem)
                pltpu.async_copy(
                    z_sh.at[idxrow], gbuf.at[buf * 8 + 2 * r + 1, dsts], sem)

    def process(cc, buf, acc):
        for c in range(8):
            pltpu.make_async_copy(
                f_ref.at[pl.ds(0, CHUNK)], gbuf.at[buf * 8 + c], sem).wait()

        @pl.when(cc + 1 < NCHUNK)
        def _prefetch():
            load_and_fire(cc + 1, 1 - buf)

        def unpack(r, sl):
            xy = lax.bitcast_convert_type(gbuf[buf * 8 + 2 * r, sl],
                                          jnp.int32)
            px = lax.bitcast_convert_type(xy & _HI_MASK, jnp.float32)
            py = lax.bitcast_convert_type(lax.shift_left(xy, 16), jnp.float32)
            pz = gbuf[buf * 8 + 2 * r + 1, sl]
            return px, py, pz

        def group_body(g, acc):
            base = g * L
            sl = pl.ds(base, L)
            p0x, p0y, p0z = unpack(0, sl)
            p1x, p1y, p1z = unpack(1, sl)
            p2x, p2y, p2z = unpack(2, sl)
            p3x, p3y, p3z = unpack(3, sl)

            v1x = p0x - p1x
            v1y = p0y - p1y
            v1z = p0z - p1z
            v2x = p2x - p1x
            v2y = p2y - p1y
            v2z = p2z - p1z
            v3x = p2x - p3x
            v3y = p2y - p3y
            v3z = p2z - p3z

            c12x = v1y * v2z - v1z * v2y
            c12y = v1z * v2x - v1x * v2z
            c12z = v1x * v2y - v1y * v2x
            c23x = v2y * v3z - v2z * v3y
            c23y = v2z * v3x - v2x * v3z
            c23z = v2x * v3y - v2y * v3x

            a2 = c12x * c12x + c12y * c12y + c12z * c12z
            b2 = c23x * c23x + c23y * c23y + c23z * c23z
            dd = c12x * c23x + c12y * c23y + c12z * c23z
            tt = v1x * c23x + v1y * c23y + v1z * c23z

            q = jnp.maximum(a2 * b2, np.float32(1e-24))
            r = _rsqrt(q)
            c = jnp.clip(dd * r, np.float32(-1.0), np.float32(1.0))
            om = f1 - c * c
            sm = om * _rsqrt(jnp.maximum(om, np.float32(1e-30)))
            s = jnp.where(tt < np.float32(0.0), -sm, sm)

            fv = f_v[buf, sl]
            pv = per_v[buf, sl]
            phv = ph_v[buf, sl]

            t = phv - _HALF_PI
            t2 = t * t
            sp = np.float32(_SIN[4])
            for cf in (_SIN[3], _SIN[2], _SIN[1], _SIN[0]):
                sp = sp * t2 + np.float32(cf)
            sp = sp * t
            cp = np.float32(_COS[5])
            for cf in (_COS[4], _COS[3], _COS[2], _COS[1], _COS[0]):
                cp = cp * t2 + np.float32(cf)
            cpsi = -sp
            spsi = cp

            cn, sn = c, s
            ck, sk = c, s
            for kk2 in range(2, 7):
                ck, sk = ck * c - sk * s, sk * c + ck * s
                sel = pv == np.float32(kk2)
                cn = jnp.where(sel, ck, cn)
                sn = jnp.where(sel, sk, sn)

            val = fv * (f1 + cn * cpsi + sn * spsi)
            return acc + val

        return lax.fori_loop(0, GROUPS, group_body, acc, unroll=False)

    load_and_fire(0, 0)

    def pair_body(p, acc):
        acc = process(2 * p, 0, acc)
        acc = process(2 * p + 1, 1, acc)
        return acc

    acc = lax.fori_loop(0, NCHUNK // 2, pair_body,
                        jnp.zeros((L,), jnp.float32), unroll=False)
    acc_v[...] = acc
    pltpu.sync_copy(acc_v, out_ref.at[wid])


_sc_call = functools.partial(
    pl.kernel,
    out_type=jax.ShapeDtypeStruct((NW, L), jnp.float32),
    mesh=plsc.VectorSubcoreMesh(core_axis_name="c", subcore_axis_name="s",
                                num_cores=NC, num_subcores=NS),
    scratch_types=[
        pltpu.VMEM((2 * 4 * NBLK_CHUNK, BLK), jnp.int32),
        pltpu.VMEM((2, CHUNK), jnp.float32),
        pltpu.VMEM((2, CHUNK), jnp.float32),
        pltpu.VMEM((2, CHUNK), jnp.float32),
        pltpu.VMEM((2 * 8, CHUNK), jnp.float32),
        pltpu.VMEM((L,), jnp.float32),
        pltpu.VMEM((ASLICE,), jnp.float32),
        pltpu.VMEM_SHARED((ATAB,), jnp.float32),
        pltpu.VMEM_SHARED((ATAB,), jnp.float32),
        pltpu.SemaphoreType.DMA,
    ],
)(_sc_body)


@jax.jit
def kernel(coords, i, j, k, l, force, period, phase):
    xi = lax.bitcast_convert_type(coords[:, 0], jnp.int32)
    yi = lax.bitcast_convert_type(coords[:, 1], jnp.int32)
    xy = ((xi + 0x8000) & _HI_MASK) | lax.shift_right_logical(
        yi + 0x8000, 16)
    xyf = lax.bitcast_convert_type(xy, jnp.float32)
    apad = ATAB - N_ATOMS
    xyt = jnp.concatenate([xyf, jnp.zeros((apad,), jnp.float32)])
    zt = jnp.concatenate([coords[:, 2], jnp.zeros((apad,), jnp.float32)])

    pad = NPAD - N_DIH

    def pad_idx(x):
        x = jnp.concatenate([x.astype(jnp.int32), jnp.zeros((pad,), jnp.int32)])
        return x.reshape(NPAD // BLK, BLK)

    i2, j2, k2, l2 = pad_idx(i), pad_idx(j), pad_idx(k), pad_idx(l)
    f2 = jnp.concatenate([force, jnp.zeros((pad,), jnp.float32)])
    p2 = jnp.concatenate([period, jnp.ones((pad,), jnp.float32)])
    ph2 = jnp.concatenate([phase, jnp.zeros((pad,), jnp.float32)])
    partials = _sc_call(xyt, zt, i2, j2, k2, l2, f2, p2, ph2)
    return jnp.sum(partials)


# speedup vs baseline: 11.4381x; 1.1813x over previous
import functools

import jax
import jax.numpy as jnp
import numpy as np
from jax import lax
from jax.experimental import pallas as pl
from jax.experimental.pallas import tpu as pltpu
from jax.experimental.pallas import tpu_sc as plsc

N_ATOMS = 100000
N_DIH = 500000
NC = 2
NS = 16
L = 16
NW = NC * NS
PER_W = 16384
NPAD = NW * PER_W
BLK = 128
NBLK_CHUNK = 8
CHUNK = BLK * NBLK_CHUNK
NCHUNK = PER_W // CHUNK
GROUPS = CHUNK // L
ATAB = 100096
ASLICE = ATAB // NS

_SIN = (1.0, -1.6666651e-01, 8.3329640e-03, -1.9804748e-04, 2.5980951e-06)
_COS = (1.0, -0.5, 4.1666642e-02, -1.3888433e-03, 2.4763767e-05, -2.6114949e-07)
_HALF_PI = np.float32(1.5707964)
_HI_MASK = np.int32(-65536)


def _rsqrt(q):
    xi = lax.bitcast_convert_type(q, jnp.int32)
    yi = jnp.int32(0x5F3759DF) - lax.shift_right_logical(xi, 1)
    y = lax.bitcast_convert_type(yi, jnp.float32)
    h = q * np.float32(0.5)
    y = y * (np.float32(1.5) - h * y * y)
    y = y * (np.float32(1.5) - h * y * y)
    return y


def _sc_body(xy_ref, z_ref, i_ref, j_ref, k_ref, l_ref, f_ref, per_ref,
             ph_ref, out_ref, raw_v, f_v, per_v, ph_v, gbuf, acc_v, stage_v,
             xy_sh, z_sh, sem, semb0, semb1):
    cidx = lax.axis_index("c")
    sidx = lax.axis_index("s")
    wid = sidx * NC + cidx
    blk0 = wid * (PER_W // BLK)
    el0 = wid * PER_W
    f1 = np.float32(1.0)

    s0 = sidx * ASLICE
    pltpu.sync_copy(xy_ref.at[pl.ds(s0, ASLICE)], stage_v)
    pltpu.sync_copy(stage_v, xy_sh.at[pl.ds(s0, ASLICE)])
    pltpu.sync_copy(z_ref.at[pl.ds(s0, ASLICE)], stage_v)
    pltpu.sync_copy(stage_v, z_sh.at[pl.ds(s0, ASLICE)])
    plsc.subcore_barrier()

    def stage(cc, s):
        row0 = blk0 + cc * NBLK_CHUNK
        e0 = el0 + cc * CHUNK
        semb = semb0 if s % 2 == 0 else semb1
        for r, ref in enumerate((i_ref, j_ref, k_ref, l_ref)):
            pltpu.async_copy(
                ref.at[pl.ds(row0, NBLK_CHUNK)],
                raw_v.at[pl.ds((s * 4 + r) * NBLK_CHUNK, NBLK_CHUNK)], semb)
        pltpu.async_copy(f_ref.at[pl.ds(e0, CHUNK)], f_v.at[s], semb)
        pltpu.async_copy(per_ref.at[pl.ds(e0, CHUNK)], per_v.at[s], semb)
        pltpu.async_copy(ph_ref.at[pl.ds(e0, CHUNK)], ph_v.at[s], semb)

    def fire(s, buf):
        semb = semb0 if s % 2 == 0 else semb1
        for r in range(4):
            pltpu.make_async_copy(
                i_ref.at[pl.ds(0, NBLK_CHUNK)],
                raw_v.at[pl.ds((s * 4 + r) * NBLK_CHUNK, NBLK_CHUNK)],
                semb).wait()
        for pref in (f_v, per_v, ph_v):
            pltpu.make_async_copy(
                f_ref.at[pl.ds(0, CHUNK)], pref.at[s], semb).wait()

        @pl.loop(0, NBLK_CHUNK)
        def _fire(b):
            dsts = pl.ds(b * BLK, BLK)
            for r in range(4):
                idxrow = raw_v.at[(s * 4 + r) * NBLK_CHUNK + b]
                pltpu.async_copy(
                    xy_sh.at[idxrow], gbuf.at[buf * 8 + 2 * r, dsts], sem)
                pltpu.async_copy(
                    z_sh.at[idxrow], gbuf.at[buf * 8 + 2 * r + 1, dsts], sem)

    def process(cc, slot, acc):
        buf = slot % 2
        for c in range(8):
            pltpu.make_async_copy(
                f_ref.at[pl.ds(0, CHUNK)], gbuf.at[buf * 8 + c], sem).wait()

        @pl.when(cc + 1 < NCHUNK)
        def _fire_next():
            fire((slot + 1) % 4, 1 - buf)

        @pl.when(cc + 2 < NCHUNK)
        def _stage_next():
            stage(cc + 2, (slot + 2) % 4)

        def unpack(r, sl):
            xy = lax.bitcast_convert_type(gbuf[buf * 8 + 2 * r, sl],
                                          jnp.int32)
            px = lax.bitcast_convert_type(xy & _HI_MASK, jnp.float32)
            py = lax.bitcast_convert_type(lax.shift_left(xy, 16), jnp.float32)
            pz = gbuf[buf * 8 + 2 * r + 1, sl]
            return px, py, pz

        def group_body(g, acc):
            base = g * L
            sl = pl.ds(base, L)
            p0x, p0y, p0z = unpack(0, sl)
            p1x, p1y, p1z = unpack(1, sl)
            p2x, p2y, p2z = unpack(2, sl)
            p3x, p3y, p3z = unpack(3, sl)

            v1x = p0x - p1x
            v1y = p0y - p1y
            v1z = p0z - p1z
            v2x = p2x - p1x
            v2y = p2y - p1y
            v2z = p2z - p1z
            v3x = p2x - p3x
            v3y = p2y - p3y
            v3z = p2z - p3z

            c12x = v1y * v2z - v1z * v2y
            c12y = v1z * v2x - v1x * v2z
            c12z = v1x * v2y - v1y * v2x
            c23x = v2y * v3z - v2z * v3y
            c23y = v2z * v3x - v2x * v3z
            c23z = v2x * v3y - v2y * v3x

            a2 = c12x * c12x + c12y * c12y + c12z * c12z
            b2 = c23x * c23x + c23y * c23y + c23z * c23z
            dd = c12x * c23x + c12y * c23y + c12z * c23z
            tt = v1x * c23x + v1y * c23y + v1z * c23z

            q = jnp.maximum(a2 * b2, np.float32(1e-24))
            r = _rsqrt(q)
            c = jnp.clip(dd * r, np.float32(-1.0), np.float32(1.0))
            om = f1 - c * c
            sm = om * _rsqrt(jnp.maximum(om, np.float32(1e-30)))
            s = jnp.where(tt < np.float32(0.0), -sm, sm)

            fv = f_v[slot, sl]
            pv = per_v[slot, sl]
            phv = ph_v[slot, sl]

            t = phv - _HALF_PI
            t2 = t * t
            sp = np.float32(_SIN[4])
            for cf in (_SIN[3], _SIN[2], _SIN[1], _SIN[0]):
                sp = sp * t2 + np.float32(cf)
            sp = sp * t
            cp = np.float32(_COS[5])
            for cf in (_COS[4], _COS[3], _COS[2], _COS[1], _COS[0]):
                cp = cp * t2 + np.float32(cf)
            cpsi = -sp
            spsi = cp

            cn, sn = c, s
            ck, sk = c, s
            for kk2 in range(2, 7):
                ck, sk = ck * c - sk * s, sk * c + ck * s
                sel = pv == np.float32(kk2)
                cn = jnp.where(sel, ck, cn)
                sn = jnp.where(sel, sk, sn)

            val = fv * (f1 + cn * cpsi + sn * spsi)
            return acc + val

        return lax.fori_loop(0, GROUPS, group_body, acc, unroll=False)

    stage(0, 0)
    fire(0, 0)
    stage(1, 1)

    def quad_body(p, acc):
        acc = process(4 * p, 0, acc)
        acc = process(4 * p + 1, 1, acc)
        acc = process(4 * p + 2, 2, acc)
        acc = process(4 * p + 3, 3, acc)
        return acc

    acc = lax.fori_loop(0, NCHUNK // 4, quad_body,
                        jnp.zeros((L,), jnp.float32), unroll=False)
    acc_v[...] = acc
    pltpu.sync_copy(acc_v, out_ref.at[wid])


_sc_call = functools.partial(
    pl.kernel,
    out_type=jax.ShapeDtypeStruct((NW, L), jnp.float32),
    mesh=plsc.VectorSubcoreMesh(core_axis_name="c", subcore_axis_name="s",
                                num_cores=NC, num_subcores=NS),
    scratch_types=[
        pltpu.VMEM((4 * 4 * NBLK_CHUNK, BLK), jnp.int32),
        pltpu.VMEM((4, CHUNK), jnp.float32),
        pltpu.VMEM((4, CHUNK), jnp.float32),
        pltpu.VMEM((4, CHUNK), jnp.float32),
        pltpu.VMEM((2 * 8, CHUNK), jnp.float32),
        pltpu.VMEM((L,), jnp.float32),
        pltpu.VMEM((ASLICE,), jnp.float32),
        pltpu.VMEM_SHARED((ATAB,), jnp.float32),
        pltpu.VMEM_SHARED((ATAB,), jnp.float32),
        pltpu.SemaphoreType.DMA,
        pltpu.SemaphoreType.DMA,
        pltpu.SemaphoreType.DMA,
    ],
)(_sc_body)


@jax.jit
def kernel(coords, i, j, k, l, force, period, phase):
    xi = lax.bitcast_convert_type(coords[:, 0], jnp.int32)
    yi = lax.bitcast_convert_type(coords[:, 1], jnp.int32)
    xy = ((xi + 0x8000) & _HI_MASK) | lax.shift_right_logical(
        yi + 0x8000, 16)
    xyf = lax.bitcast_convert_type(xy, jnp.float32)
    apad = ATAB - N_ATOMS
    xyt = jnp.concatenate([xyf, jnp.zeros((apad,), jnp.float32)])
    zt = jnp.concatenate([coords[:, 2], jnp.zeros((apad,), jnp.float32)])

    pad = NPAD - N_DIH

    def pad_idx(x):
        x = jnp.concatenate([x.astype(jnp.int32), jnp.zeros((pad,), jnp.int32)])
        return x.reshape(NPAD // BLK, BLK)

    i2, j2, k2, l2 = pad_idx(i), pad_idx(j), pad_idx(k), pad_idx(l)
    f2 = jnp.concatenate([force, jnp.zeros((pad,), jnp.float32)])
    p2 = jnp.concatenate([period, jnp.ones((pad,), jnp.float32)])
    ph2 = jnp.concatenate([phase, jnp.zeros((pad,), jnp.float32)])
    partials = _sc_call(xyt, zt, i2, j2, k2, l2, f2, p2, ph2)
    return jnp.sum(partials)
